# baseline (speedup 1.0000x reference)
import functools

import jax
import jax.numpy as jnp
from jax import lax
from jax.experimental import pallas as pl
from jax.experimental.pallas import tpu as pltpu
from jax.experimental.pallas import tpu_sc as plsc

N_USERS = 10000
N_NODES = 50000
E_EDGES = 800000
D = 64
DH = 32
B = 1024

NSUB = 16
GRP = 128
NCHUNK = 10
GC = 42
E_PAD = NSUB * NCHUNK * GC * GRP
ROWS_PER_SUB = 3136
N_PAD = NSUB * ROWS_PER_SUB
ZROWS = 112

_MESH = plsc.VectorSubcoreMesh(core_axis_name="c", subcore_axis_name="s")


def _agg_body(feat, tgt, nbr, val, out, idx_t, idx_n, vals, rows0, rows1,
              rows2, acc, gs0, gs1, gs2):
    c = lax.axis_index("c")
    s = lax.axis_index("s")
    rb = (rows0, rows1, rows2)
    gs = (gs0, gs1, gs2)

    def _zrow(i, _):
        rows0[i, pl.ds(0, 16)] = jnp.zeros((16,), jnp.float32)
        rows0[i, pl.ds(16, 16)] = jnp.zeros((16,), jnp.float32)
        return 0
    lax.fori_loop(0, GRP, _zrow, 0)
    base = s * ROWS_PER_SUB
    for k in range(ROWS_PER_SUB // ZROWS):
        pltpu.sync_copy(rows0.at[pl.ds(0, ZROWS)],
                        acc.at[pl.ds(base + k * ZROWS, ZROWS)])
    plsc.subcore_barrier()

    def _scale(rows, g):
        gbase = g * GRP
        for blk in range(GRP // 16):
            vv = vals[pl.ds(gbase + blk * 16, 16)]
            for lane in range(16):
                e = blk * 16 + lane
                m = jnp.full((16,), vv[lane], jnp.float32)
                rows[e, pl.ds(0, 16)] = rows[e, pl.ds(0, 16)] * m
                rows[e, pl.ds(16, 16)] = rows[e, pl.ds(16, 16)] * m

    def _gstart(g, b):
        pltpu.async_copy(feat.at[c].at[idx_n.at[pl.ds(g * GRP, GRP)]],
                         rb[b], gs[b])

    def _gwait(g, b):
        pltpu.make_async_copy(feat.at[c].at[idx_n.at[pl.ds(g * GRP, GRP)]],
                              rb[b], gs[b]).wait()

    def _sscat(g, b):
        pltpu.sync_copy(rb[b], acc.at[idx_t.at[g]], add=True)

    def _chunk(ch, _):
        off = (s * NCHUNK + ch) * (GC * GRP)
        pltpu.sync_copy(tgt.at[s, ch], idx_t)
        pltpu.sync_copy(nbr.at[pl.ds(off, GC * GRP)], idx_n)
        pltpu.sync_copy(val.at[pl.ds(off, GC * GRP)], vals)

        _gstart(0, 0)
        _gstart(1, 1)

        def _tri(i, __):
            for d in range(3):
                g = 3 * i + d
                b = d % 3
                _gstart(g + 2, (b + 2) % 3)
                _gwait(g, b)
                _scale(rb[b], g)
                _sscat(g, b)
            return 0
        lax.fori_loop(0, (GC - 3) // 3, _tri, 0)

        for g in (GC - 3, GC - 2, GC - 1):
            b = g % 3
            if g == GC - 3:
                _gstart(g + 2, (b + 2) % 3)
            _gwait(g, b)
            _scale(rb[b], g)
            _sscat(g, b)
        return 0
    lax.fori_loop(0, NCHUNK, _chunk, 0)

    plsc.subcore_barrier()
    pltpu.sync_copy(acc.at[pl.ds(base, ROWS_PER_SUB)],
                    out.at[c].at[pl.ds(base, ROWS_PER_SUB)])


_agg = pl.kernel(
    _agg_body,
    out_type=jax.ShapeDtypeStruct((2, N_PAD, DH), jnp.float32),
    mesh=_MESH,
    scratch_types=[
        pltpu.VMEM((GC, GRP), jnp.int32),
        pltpu.VMEM((GC * GRP,), jnp.int32),
        pltpu.VMEM((GC * GRP,), jnp.float32),
        pltpu.VMEM((GRP, DH), jnp.float32),
        pltpu.VMEM((GRP, DH), jnp.float32),
        pltpu.VMEM((GRP, DH), jnp.float32),
        pltpu.VMEM_SHARED((N_PAD, DH), jnp.float32),
        pltpu.SemaphoreType.DMA,
        pltpu.SemaphoreType.DMA,
        pltpu.SemaphoreType.DMA,
    ],
    compiler_params=pltpu.CompilerParams(use_tc_tiling_on_sc=False),
)


def _dense0_body(f_ref, h0_ref, h1_ref, w1_ref, w2_ref, b1_ref, b2_ref, out_ref):
    f = f_ref[...]
    hn = jnp.concatenate([h0_ref[0], h1_ref[0]], axis=1)
    sm = f + hn
    pr = f * hn
    z = (jnp.dot(sm, w1_ref[...], preferred_element_type=jnp.float32)
         + jnp.dot(pr, w2_ref[...], preferred_element_type=jnp.float32)
         + b1_ref[...] + b2_ref[...])
    hh = jnp.where(z >= 0, z, 0.01 * z)
    nrm = jnp.sqrt(jnp.sum(hh * hh, axis=1, keepdims=True))
    hh = hh / jnp.maximum(nrm, 1e-12)
    out_ref[0] = hh[:, :DH]
    out_ref[1] = hh[:, DH:]


BR = 5000


def _dense0(e0_full, hn, w1, w2, b1, b2):
    return pl.pallas_call(
        _dense0_body,
        grid=(N_NODES // BR,),
        in_specs=[
            pl.BlockSpec((BR, D), lambda i: (i, 0)),
            pl.BlockSpec((1, BR, DH), lambda i: (0, i, 0)),
            pl.BlockSpec((1, BR, DH), lambda i: (1, i, 0)),
            pl.BlockSpec((D, D), lambda i: (0, 0)),
            pl.BlockSpec((D, D), lambda i: (0, 0)),
            pl.BlockSpec((1, D), lambda i: (0, 0)),
            pl.BlockSpec((1, D), lambda i: (0, 0)),
        ],
        out_specs=pl.BlockSpec((2, BR, DH), lambda i: (0, i, 0)),
        out_shape=jax.ShapeDtypeStruct((2, N_NODES, DH), jnp.float32),
    )(e0_full, hn, hn, w1, w2, b1, b2)


def _dense1_body(f0_ref, f1_ref, h0_ref, h1_ref, w1_ref, w2_ref, b1_ref,
                 b2_ref, out_ref):
    w1 = w1_ref[...]
    w2 = w2_ref[...]
    f0 = f0_ref[0]
    f1 = f1_ref[0]
    h0 = h0_ref[0]
    h1 = h1_ref[0]
    z = (jnp.dot(f0 + h0, w1[:DH], preferred_element_type=jnp.float32)
         + jnp.dot(f1 + h1, w1[DH:], preferred_element_type=jnp.float32)
         + jnp.dot(f0 * h0, w2[:DH], preferred_element_type=jnp.float32)
         + jnp.dot(f1 * h1, w2[DH:], preferred_element_type=jnp.float32)
         + b1_ref[...] + b2_ref[...])
    hh = jnp.where(z >= 0, z, 0.01 * z)
    nrm = jnp.sqrt(jnp.sum(hh * hh, axis=1, keepdims=True))
    out_ref[...] = hh / jnp.maximum(nrm, 1e-12)


def _dense1(f1s, hn, w1, w2, b1, b2):
    return pl.pallas_call(
        _dense1_body,
        grid=(N_NODES // BR,),
        in_specs=[
            pl.BlockSpec((1, BR, DH), lambda i: (0, i, 0)),
            pl.BlockSpec((1, BR, DH), lambda i: (1, i, 0)),
            pl.BlockSpec((1, BR, DH), lambda i: (0, i, 0)),
            pl.BlockSpec((1, BR, DH), lambda i: (1, i, 0)),
            pl.BlockSpec((D, DH), lambda i: (0, 0)),
            pl.BlockSpec((D, DH), lambda i: (0, 0)),
            pl.BlockSpec((1, DH), lambda i: (0, 0)),
            pl.BlockSpec((1, DH), lambda i: (0, 0)),
        ],
        out_specs=pl.BlockSpec((BR, DH), lambda i: (i, 0)),
        out_shape=jax.ShapeDtypeStruct((N_NODES, DH), jnp.float32),
    )(f1s, f1s, hn, hn, w1, w2, b1, b2)


BPS = B // 32


def _score_body(uid, pid, e0s, e1s, e2, out, uix, pix, ub, pb, accv, obuf):
    c = lax.axis_index("c")
    s = lax.axis_index("s")
    w = s * 2 + c
    base = w * BPS
    pltpu.sync_copy(uid.at[pl.ds(base, BPS)], uix)
    pltpu.sync_copy(pid.at[pl.ds(base, BPS)], pix)
    for j in range(BPS // 16):
        pix[pl.ds(16 * j, 16)] = pix[pl.ds(16 * j, 16)] + N_USERS
    for r in range(BPS):
        accv[r] = jnp.zeros((16,), jnp.float32)
    parts = [(e0s, 0), (e0s, 1), (e1s, 0), (e1s, 1), (e2, None)]
    for tab, cc in parts:
        t = tab if cc is None else tab.at[cc]
        pltpu.sync_copy(t.at[uix], ub)
        pltpu.sync_copy(t.at[pix], pb)
        for r in range(BPS):
            accv[r] = (accv[r]
                       + ub[r, pl.ds(0, 16)] * pb[r, pl.ds(0, 16)]
                       + ub[r, pl.ds(16, 16)] * pb[r, pl.ds(16, 16)])
    for j in range(BPS // 16):
        obuf[pl.ds(16 * j, 16)] = jnp.zeros((16,), jnp.float32)
    lanes = lax.iota(jnp.int32, 16)

    dnums = lax.GatherDimensionNumbers(
        offset_dims=(), collapsed_slice_dims=(0,), start_index_map=(0,))

    def _lane_sum(x):
        for k in (8, 4, 2, 1):
            perm = lanes ^ k
            x = x + lax.gather(x, perm[:, None], dnums, (1,),
                               mode=lax.GatherScatterMode.PROMISE_IN_BOUNDS)
        return x

    for r in range(BPS):
        sc = _lane_sum(accv[r])
        j = r // 16
        ob = obuf[pl.ds(16 * j, 16)]
        obuf[pl.ds(16 * j, 16)] = jnp.where(lanes == (r % 16), sc, ob)
    pltpu.sync_copy(obuf, out.at[pl.ds(base, BPS)])


_score = pl.kernel(
    _score_body,
    out_type=jax.ShapeDtypeStruct((B,), jnp.float32),
    mesh=_MESH,
    scratch_types=[
        pltpu.VMEM((BPS,), jnp.int32),
        pltpu.VMEM((BPS,), jnp.int32),
        pltpu.VMEM((BPS, DH), jnp.float32),
        pltpu.VMEM((BPS, DH), jnp.float32),
        pltpu.VMEM((BPS, 16), jnp.float32),
        pltpu.VMEM((BPS,), jnp.float32),
    ],
    compiler_params=pltpu.CompilerParams(use_tc_tiling_on_sc=False),
)


def kernel(edge_target, edge_neighbor, edge_values, user_ids, pos_item_ids,
           user_embed, entity_embed,
           W1_0, b1_0, W2_0, b2_0, W1_1, b1_1, W2_1, b2_1):
    pad = E_PAD - E_EDGES
    shape = (NSUB, NCHUNK, GC, GRP)
    tgt = jnp.concatenate(
        [edge_target.astype(jnp.int32), jnp.zeros((pad,), jnp.int32)]
    ).reshape(shape)
    nbr = jnp.concatenate(
        [edge_neighbor.astype(jnp.int32), jnp.zeros((pad,), jnp.int32)])
    val = jnp.concatenate(
        [edge_values, jnp.zeros((pad,), jnp.float32)])

    e0_full = jnp.concatenate([user_embed, entity_embed], axis=0)
    e0s = e0_full.reshape(N_NODES, 2, DH).transpose(1, 0, 2)

    hn0 = _agg(e0s, tgt, nbr, val)
    f1s = _dense0(e0_full, hn0, W1_0, W2_0,
                  b1_0.reshape(1, D), b2_0.reshape(1, D))
    hn1 = _agg(f1s, tgt, nbr, val)
    f2 = _dense1(f1s, hn1, W1_1, W2_1,
                 b1_1.reshape(1, DH), b2_1.reshape(1, DH))
    return _score(user_ids.astype(jnp.int32), pos_item_ids.astype(jnp.int32),
                  e0s, f1s, f2)


# speedup vs baseline: 2.0384x; 2.0384x over previous
import functools

import jax
import jax.numpy as jnp
from jax import lax
from jax.experimental import pallas as pl
from jax.experimental.pallas import tpu as pltpu
from jax.experimental.pallas import tpu_sc as plsc

N_USERS = 10000
N_NODES = 50000
E_EDGES = 800000
D = 64
DH = 32
B = 1024

NSUB = 16
GRP = 128
NCHUNK = 14
GC = 28
E_PAD = NSUB * NCHUNK * GC * GRP
ROWS_PER_SUB = 3136
N_PAD = NSUB * ROWS_PER_SUB
ZROWS = 112

_MESH = plsc.VectorSubcoreMesh(core_axis_name="c", subcore_axis_name="s")


def _agg_body(feat, tgt, nbr, val, out, idx_t, idx_n, vals, rows0, rows1,
              zbuf, acc, gsem0, gsem1):
    c = lax.axis_index("c")
    s = lax.axis_index("s")

    def _zrow(i, _):
        zbuf[i, pl.ds(0, 16)] = jnp.zeros((16,), jnp.float32)
        zbuf[i, pl.ds(16, 16)] = jnp.zeros((16,), jnp.float32)
        return 0
    lax.fori_loop(0, ZROWS, _zrow, 0)
    base = s * ROWS_PER_SUB
    for k in range(ROWS_PER_SUB // ZROWS):
        pltpu.sync_copy(zbuf, acc.at[pl.ds(base + k * ZROWS, ZROWS)])
    plsc.subcore_barrier()

    def _scale_scatter(rows, g):
        gbase = g * GRP
        for blk in range(GRP // 16):
            vv = vals[pl.ds(gbase + blk * 16, 16)]
            for lane in range(16):
                e = blk * 16 + lane
                m = jnp.full((16,), vv[lane], jnp.float32)
                rows[e, pl.ds(0, 16)] = rows[e, pl.ds(0, 16)] * m
                rows[e, pl.ds(16, 16)] = rows[e, pl.ds(16, 16)] * m
        pltpu.sync_copy(rows, acc.at[idx_t.at[g]], add=True)

    def _gather(g, rows, sem):
        pltpu.async_copy(feat.at[c].at[idx_n.at[pl.ds(g * GRP, GRP)]],
                         rows, sem)

    def _gwait(g, rows, sem):
        pltpu.make_async_copy(feat.at[c].at[idx_n.at[pl.ds(g * GRP, GRP)]],
                              rows, sem).wait()

    def _chunk(ch, _):
        off = (s * NCHUNK + ch) * (GC * GRP)
        pltpu.sync_copy(tgt.at[s, ch], idx_t)
        pltpu.sync_copy(nbr.at[pl.ds(off, GC * GRP)], idx_n)
        pltpu.sync_copy(val.at[pl.ds(off, GC * GRP)], vals)

        _gather(0, rows0, gsem0)

        def _pair(i, __):
            g = 2 * i
            _gather(g + 1, rows1, gsem1)
            _gwait(g, rows0, gsem0)
            _scale_scatter(rows0, g)
            _gather(g + 2, rows0, gsem0)
            _gwait(g + 1, rows1, gsem1)
            _scale_scatter(rows1, g + 1)
            return 0
        lax.fori_loop(0, GC // 2 - 1, _pair, 0)

        gl = GC - 2
        _gather(gl + 1, rows1, gsem1)
        _gwait(gl, rows0, gsem0)
        _scale_scatter(rows0, gl)
        _gwait(gl + 1, rows1, gsem1)
        _scale_scatter(rows1, gl + 1)
        return 0
    lax.fori_loop(0, NCHUNK, _chunk, 0)

    plsc.subcore_barrier()
    pltpu.sync_copy(acc.at[pl.ds(base, ROWS_PER_SUB)],
                    out.at[c].at[pl.ds(base, ROWS_PER_SUB)])


_agg = pl.kernel(
    _agg_body,
    out_type=jax.ShapeDtypeStruct((2, N_PAD, DH), jnp.float32),
    mesh=_MESH,
    scratch_types=[
        pltpu.VMEM((GC, GRP), jnp.int32),
        pltpu.VMEM((GC * GRP,), jnp.int32),
        pltpu.VMEM((GC * GRP,), jnp.float32),
        pltpu.VMEM((GRP, DH), jnp.float32),
        pltpu.VMEM((GRP, DH), jnp.float32),
        pltpu.VMEM((ZROWS, DH), jnp.float32),
        pltpu.VMEM_SHARED((N_PAD, DH), jnp.float32),
        pltpu.SemaphoreType.DMA,
        pltpu.SemaphoreType.DMA,
    ],
    compiler_params=pltpu.CompilerParams(use_tc_tiling_on_sc=False),
)


def _dense0_body(f_ref, h0_ref, h1_ref, w1_ref, w2_ref, b1_ref, b2_ref, out_ref):
    f = f_ref[...]
    hn = jnp.concatenate([h0_ref[0], h1_ref[0]], axis=1)
    sm = f + hn
    pr = f * hn
    z = (jnp.dot(sm, w1_ref[...], preferred_element_type=jnp.float32)
         + jnp.dot(pr, w2_ref[...], preferred_element_type=jnp.float32)
         + b1_ref[...] + b2_ref[...])
    hh = jnp.where(z >= 0, z, 0.01 * z)
    nrm = jnp.sqrt(jnp.sum(hh * hh, axis=1, keepdims=True))
    hh = hh / jnp.maximum(nrm, 1e-12)
    out_ref[0] = hh[:, :DH]
    out_ref[1] = hh[:, DH:]


BR = 5000


def _dense0(e0_full, hn, w1, w2, b1, b2):
    return pl.pallas_call(
        _dense0_body,
        grid=(N_NODES // BR,),
        in_specs=[
            pl.BlockSpec((BR, D), lambda i: (i, 0)),
            pl.BlockSpec((1, BR, DH), lambda i: (0, i, 0)),
            pl.BlockSpec((1, BR, DH), lambda i: (1, i, 0)),
            pl.BlockSpec((D, D), lambda i: (0, 0)),
            pl.BlockSpec((D, D), lambda i: (0, 0)),
            pl.BlockSpec((1, D), lambda i: (0, 0)),
            pl.BlockSpec((1, D), lambda i: (0, 0)),
        ],
        out_specs=pl.BlockSpec((2, BR, DH), lambda i: (0, i, 0)),
        out_shape=jax.ShapeDtypeStruct((2, N_NODES, DH), jnp.float32),
    )(e0_full, hn, hn, w1, w2, b1, b2)


def _dense1_body(f0_ref, f1_ref, h0_ref, h1_ref, w1_ref, w2_ref, b1_ref,
                 b2_ref, out_ref):
    w1 = w1_ref[...]
    w2 = w2_ref[...]
    f0 = f0_ref[0]
    f1 = f1_ref[0]
    h0 = h0_ref[0]
    h1 = h1_ref[0]
    z = (jnp.dot(f0 + h0, w1[:DH], preferred_element_type=jnp.float32)
         + jnp.dot(f1 + h1, w1[DH:], preferred_element_type=jnp.float32)
         + jnp.dot(f0 * h0, w2[:DH], preferred_element_type=jnp.float32)
         + jnp.dot(f1 * h1, w2[DH:], preferred_element_type=jnp.float32)
         + b1_ref[...] + b2_ref[...])
    hh = jnp.where(z >= 0, z, 0.01 * z)
    nrm = jnp.sqrt(jnp.sum(hh * hh, axis=1, keepdims=True))
    out_ref[...] = hh / jnp.maximum(nrm, 1e-12)


def _dense1(f1s, hn, w1, w2, b1, b2):
    return pl.pallas_call(
        _dense1_body,
        grid=(N_NODES // BR,),
        in_specs=[
            pl.BlockSpec((1, BR, DH), lambda i: (0, i, 0)),
            pl.BlockSpec((1, BR, DH), lambda i: (1, i, 0)),
            pl.BlockSpec((1, BR, DH), lambda i: (0, i, 0)),
            pl.BlockSpec((1, BR, DH), lambda i: (1, i, 0)),
            pl.BlockSpec((D, DH), lambda i: (0, 0)),
            pl.BlockSpec((D, DH), lambda i: (0, 0)),
            pl.BlockSpec((1, DH), lambda i: (0, 0)),
            pl.BlockSpec((1, DH), lambda i: (0, 0)),
        ],
        out_specs=pl.BlockSpec((BR, DH), lambda i: (i, 0)),
        out_shape=jax.ShapeDtypeStruct((N_NODES, DH), jnp.float32),
    )(f1s, f1s, hn, hn, w1, w2, b1, b2)


BPS = B // 32


def _score_body(uid, pid, e0s, e1s, e2, out, uix, pix, ub, pb, accv, obuf):
    c = lax.axis_index("c")
    s = lax.axis_index("s")
    w = s * 2 + c
    base = w * BPS
    pltpu.sync_copy(uid.at[pl.ds(base, BPS)], uix)
    pltpu.sync_copy(pid.at[pl.ds(base, BPS)], pix)
    for j in range(BPS // 16):
        pix[pl.ds(16 * j, 16)] = pix[pl.ds(16 * j, 16)] + N_USERS
    for r in range(BPS):
        accv[r] = jnp.zeros((16,), jnp.float32)
    parts = [(e0s, 0), (e0s, 1), (e1s, 0), (e1s, 1), (e2, None)]
    for tab, cc in parts:
        t = tab if cc is None else tab.at[cc]
        pltpu.sync_copy(t.at[uix], ub)
        pltpu.sync_copy(t.at[pix], pb)
        for r in range(BPS):
            accv[r] = (accv[r]
                       + ub[r, pl.ds(0, 16)] * pb[r, pl.ds(0, 16)]
                       + ub[r, pl.ds(16, 16)] * pb[r, pl.ds(16, 16)])
    for j in range(BPS // 16):
        obuf[pl.ds(16 * j, 16)] = jnp.zeros((16,), jnp.float32)
    lanes = lax.iota(jnp.int32, 16)

    dnums = lax.GatherDimensionNumbers(
        offset_dims=(), collapsed_slice_dims=(0,), start_index_map=(0,))

    def _lane_sum(x):
        for k in (8, 4, 2, 1):
            perm = lanes ^ k
            x = x + lax.gather(x, perm[:, None], dnums, (1,),
                               mode=lax.GatherScatterMode.PROMISE_IN_BOUNDS)
        return x

    for r in range(BPS):
        sc = _lane_sum(accv[r])
        j = r // 16
        ob = obuf[pl.ds(16 * j, 16)]
        obuf[pl.ds(16 * j, 16)] = jnp.where(lanes == (r % 16), sc, ob)
    pltpu.sync_copy(obuf, out.at[pl.ds(base, BPS)])


_score = pl.kernel(
    _score_body,
    out_type=jax.ShapeDtypeStruct((B,), jnp.float32),
    mesh=_MESH,
    scratch_types=[
        pltpu.VMEM((BPS,), jnp.int32),
        pltpu.VMEM((BPS,), jnp.int32),
        pltpu.VMEM((BPS, DH), jnp.float32),
        pltpu.VMEM((BPS, DH), jnp.float32),
        pltpu.VMEM((BPS, 16), jnp.float32),
        pltpu.VMEM((BPS,), jnp.float32),
    ],
    compiler_params=pltpu.CompilerParams(use_tc_tiling_on_sc=False),
)


def kernel(edge_target, edge_neighbor, edge_values, user_ids, pos_item_ids,
           user_embed, entity_embed,
           W1_0, b1_0, W2_0, b2_0, W1_1, b1_1, W2_1, b2_1):
    pad = E_PAD - E_EDGES
    shape = (NSUB, NCHUNK, GC, GRP)
    tgt = jnp.concatenate(
        [edge_target.astype(jnp.int32), jnp.zeros((pad,), jnp.int32)]
    ).reshape(shape)
    nbr = jnp.concatenate(
        [edge_neighbor.astype(jnp.int32), jnp.zeros((pad,), jnp.int32)])
    val = jnp.concatenate(
        [edge_values, jnp.zeros((pad,), jnp.float32)])

    e0_full = jnp.concatenate([user_embed, entity_embed], axis=0)
    e0s = e0_full.reshape(N_NODES, 2, DH).transpose(1, 0, 2)

    hn0 = _agg(e0s, tgt, nbr, val)
    f1s = _dense0(e0_full, hn0, W1_0, W2_0,
                  b1_0.reshape(1, D), b2_0.reshape(1, D))
    hn1 = _agg(f1s, tgt, nbr, val)
    f2 = _dense1(f1s, hn1, W1_1, W2_1,
                 b1_1.reshape(1, DH), b2_1.reshape(1, DH))
    return _score(user_ids.astype(jnp.int32), pos_item_ids.astype(jnp.int32),
                  e0s, f1s, f2)
